# baseline (speedup 1.0000x reference)
import jax
import jax.numpy as jnp
from jax.experimental import pallas as pl
from jax.experimental.pallas import tpu as pltpu

_D = 768
_E = 64
_K = 8
_A = 10.0
_R = 4096


def _gating_kernel(x_ref, w_ref, b_ref, o_ref):
    x = x_ref[...]
    w = w_ref[...]
    b = b_ref[...]
    lt = jax.lax.dot_general(
        w, x, (((1,), (1,)), ((), ())),
        preferred_element_type=jnp.float32,
    ) + b

    neg_inf = jnp.float32(-jnp.inf)
    cur = lt
    for _ in range(_K):
        m = jnp.max(cur, axis=0, keepdims=True)
        cur = jnp.where(cur < m, cur, neg_inf)
    mask = cur != neg_inf

    e = jnp.exp(lt)
    sm = e / jnp.sum(e, axis=0, keepdims=True)

    t = sm + 1.0
    t2 = t * t
    t4 = t2 * t2
    t8 = t4 * t4
    e2 = jnp.where(mask, t8 * t2, jnp.exp(_A * jnp.exp(sm) - _A))
    g = e2 / jnp.sum(e2, axis=0, keepdims=True)
    o_ref[...] = g.T


def kernel(x, W, b):
    n = x.shape[0]
    b2 = b.reshape(_E, 1)
    return pl.pallas_call(
        _gating_kernel,
        grid=(n // _R,),
        in_specs=[
            pl.BlockSpec((_R, _D), lambda i: (i, 0)),
            pl.BlockSpec((_E, _D), lambda i: (0, 0)),
            pl.BlockSpec((_E, 1), lambda i: (0, 0)),
        ],
        out_specs=pl.BlockSpec((_R, _E), lambda i: (i, 0)),
        out_shape=jax.ShapeDtypeStruct((n, _E), jnp.float32),
        compiler_params=pltpu.CompilerParams(
            dimension_semantics=("parallel",),
        ),
    )(x, W, b2)


# speedup vs baseline: 1.0006x; 1.0006x over previous
import jax
import jax.numpy as jnp
from jax.experimental import pallas as pl
from jax.experimental.pallas import tpu as pltpu

_D = 768
_E = 64
_K = 8
_A = 10.0
_R = 4096


def _gating_kernel(x_ref, w_ref, b_ref, o_ref):
    x = x_ref[...]
    w = w_ref[...]
    b = b_ref[...]
    lt = jax.lax.dot_general(
        w, x, (((1,), (1,)), ((), ())),
        preferred_element_type=jnp.float32,
    ) + b

    neg_inf = jnp.float32(-jnp.inf)
    cur = lt
    for _ in range(_K):
        m = jnp.max(cur, axis=0, keepdims=True)
        cur = jnp.where(cur < m, cur, neg_inf)
    mask = cur != neg_inf

    e = jnp.exp(lt)
    sm = e / jnp.sum(e, axis=0, keepdims=True)

    t = sm + 1.0
    t2 = t * t
    t4 = t2 * t2
    t8 = t4 * t4
    e2 = jnp.where(mask, t8 * t2, jnp.exp(_A * jnp.exp(sm) - _A))
    g = e2 / jnp.sum(e2, axis=0, keepdims=True)
    o_ref[...] = g.T


def kernel(x, W, b):
    n = x.shape[0]
    b2 = b.reshape(_E, 1)
    return pl.pallas_call(
        _gating_kernel,
        grid=(n // _R,),
        in_specs=[
            pl.BlockSpec((_R, _D), lambda i: (i, 0)),
            pl.BlockSpec((_E, _D), lambda i: (0, 0)),
            pl.BlockSpec((_E, 1), lambda i: (0, 0)),
        ],
        out_specs=pl.BlockSpec((_R, _E), lambda i: (i, 0)),
        out_shape=jax.ShapeDtypeStruct((n, _E), jnp.float32),
        compiler_params=pltpu.CompilerParams(
            dimension_semantics=("arbitrary",),
        ),
    )(x, W, b2)


# speedup vs baseline: 1.0048x; 1.0041x over previous
import jax
import jax.numpy as jnp
from jax.experimental import pallas as pl
from jax.experimental.pallas import tpu as pltpu

_D = 768
_E = 64
_K = 8
_A = 10.0
_R = 4096


def _gating_kernel(x_ref, w_ref, b_ref, o_ref):
    x = x_ref[...]
    w = w_ref[...]
    b = b_ref[...]
    lt = jax.lax.dot_general(
        w, x, (((1,), (1,)), ((), ())),
        preferred_element_type=jnp.float32,
    ) + b

    neg_inf = jnp.float32(-jnp.inf)
    cur = lt
    for _ in range(_K - 1):
        m = jnp.max(cur, axis=0, keepdims=True)
        cur = jnp.where(cur < m, cur, neg_inf)
    kth = jnp.max(cur, axis=0, keepdims=True)
    mask = cur < kth

    e = jnp.exp(lt)
    sm = e / jnp.sum(e, axis=0, keepdims=True)

    t = sm + 1.0
    t2 = t * t
    t4 = t2 * t2
    t8 = t4 * t4
    e2 = jnp.where(mask, t8 * t2, jnp.exp(_A * jnp.exp(sm) - _A))
    g = e2 / jnp.sum(e2, axis=0, keepdims=True)
    o_ref[...] = g.T


def kernel(x, W, b):
    n = x.shape[0]
    b2 = b.reshape(_E, 1)
    return pl.pallas_call(
        _gating_kernel,
        grid=(n // _R,),
        in_specs=[
            pl.BlockSpec((_R, _D), lambda i: (i, 0)),
            pl.BlockSpec((_E, _D), lambda i: (0, 0)),
            pl.BlockSpec((_E, 1), lambda i: (0, 0)),
        ],
        out_specs=pl.BlockSpec((_R, _E), lambda i: (i, 0)),
        out_shape=jax.ShapeDtypeStruct((n, _E), jnp.float32),
        compiler_params=pltpu.CompilerParams(
            dimension_semantics=("arbitrary",),
        ),
    )(x, W, b2)


# speedup vs baseline: 1.0053x; 1.0005x over previous
import jax
import jax.numpy as jnp
from jax.experimental import pallas as pl
from jax.experimental.pallas import tpu as pltpu

_D = 768
_E = 64
_K = 8
_A = 10.0
_R = 4096


def _gating_kernel(x_ref, w_ref, b_ref, o_ref):
    x = x_ref[...]
    w = w_ref[...]
    b = b_ref[...]
    lt = jax.lax.dot_general(
        w, x, (((1,), (1,)), ((), ())),
        preferred_element_type=jnp.float32,
    ) + b

    neg_inf = jnp.float32(-jnp.inf)
    cur = lt
    for _ in range(_K - 1):
        m = jnp.max(cur, axis=0, keepdims=True)
        cur = jnp.where(cur < m, cur, neg_inf)
    kth = jnp.max(cur, axis=0, keepdims=True)
    mask = lt < kth

    e = jnp.exp(lt)
    sm = e / jnp.sum(e, axis=0, keepdims=True)

    t = sm + 1.0
    t2 = t * t
    t4 = t2 * t2
    t8 = t4 * t4
    e2 = jnp.where(mask, t8 * t2, jnp.exp(_A * jnp.exp(sm) - _A))
    g = e2 / jnp.sum(e2, axis=0, keepdims=True)
    o_ref[...] = g.T


def kernel(x, W, b):
    n = x.shape[0]
    b2 = b.reshape(_E, 1)
    return pl.pallas_call(
        _gating_kernel,
        grid=(n // _R,),
        in_specs=[
            pl.BlockSpec((_R, _D), lambda i: (i, 0)),
            pl.BlockSpec((_E, _D), lambda i: (0, 0)),
            pl.BlockSpec((_E, 1), lambda i: (0, 0)),
        ],
        out_specs=pl.BlockSpec((_R, _E), lambda i: (i, 0)),
        out_shape=jax.ShapeDtypeStruct((n, _E), jnp.float32),
        compiler_params=pltpu.CompilerParams(
            dimension_semantics=("arbitrary",),
        ),
    )(x, W, b2)
